# baseline (speedup 1.0000x reference)
import jax
import jax.numpy as jnp
from jax import lax
from jax.experimental import pallas as pl
from jax.experimental.pallas import tpu as pltpu

LANE = 128
OH1 = 15
CH = 4
HW_LANES = 1024
ROW_STRIDE = 64
RF = 128
RGB_D = CH * HW_LANES
TASK_D = 80
NOUT = 64
TM = 512


def _round_up(x, m):
    return ((x + m - 1) // m) * m


def _fused_body(x_ref, idx_ref, embw_ref, embb_ref,
                w1_ref, b1_ref, t2_ref, b2_ref,
                m1_ref, bm1_ref, m2_ref, bm2_ref,
                wp1_ref, bp1_ref, wp2_ref, bp2_ref, wp3_ref, bp3_ref,
                wh1_ref, bh1_ref, wh2_ref, bh2_ref, wh3_ref, bh3_ref,
                whf_ref, bhf_ref, logit_ref, value_ref):
    bf16 = jnp.bfloat16
    f32 = jnp.float32

    def dense(x, w_ref, b_ref, relu=True, out_dtype=bf16):
        y = jnp.dot(x, w_ref[...], preferred_element_type=f32) + b_ref[...]
        if relu:
            y = jnp.maximum(y, 0.0)
        return y.astype(out_dtype)

    xb = x_ref[...].astype(bf16)
    parts = []
    for oh in range(OH1):
        off = oh * ROW_STRIDE
        lhs = jnp.concatenate(
            [xb[:, c * HW_LANES + off: c * HW_LANES + off + RF]
             for c in range(CH)],
            axis=1)
        p = jnp.dot(lhs, w1_ref[...], preferred_element_type=f32)
        p = p + b1_ref[:, oh * LANE:(oh + 1) * LANE]
        parts.append(jnp.maximum(p, 0.0).astype(bf16))
    h1 = jnp.concatenate(parts, axis=1)
    cnn = dense(h1, t2_ref, b2_ref)

    idx = idx_ref[...].reshape(TM, 1)
    onehot = (idx == lax.broadcasted_iota(jnp.int32, (TM, 64), 1)).astype(f32)
    emb = jnp.maximum(
        jnp.dot(onehot, embw_ref[...], preferred_element_type=f32)
        + embb_ref[...][None, :], 0.0)
    a1 = (jnp.dot(emb.astype(bf16), m1_ref[0:32, :],
                  preferred_element_type=f32)
          + jnp.dot(xb[:, RGB_D:RGB_D + TASK_D], m1_ref[32:112, :],
                    preferred_element_type=f32)
          + bm1_ref[...])
    a1 = jnp.maximum(a1, 0.0).astype(bf16)
    a2 = dense(a1, m2_ref, bm2_ref)

    cat = cnn + a2
    x = dense(cat, wp1_ref, bp1_ref)
    x = dense(x, wp2_ref, bp2_ref)
    x = dense(x, wp3_ref, bp3_ref)
    hh = dense(x, wh1_ref, bh1_ref)
    hh = dense(hh, wh2_ref, bh2_ref)
    hh = dense(hh, wh3_ref, bh3_ref)
    y = jnp.dot(hh, whf_ref[...], preferred_element_type=f32) + bhf_ref[...]
    logit_ref[...] = y[:, :NOUT]
    value_ref[...] = y[:, NOUT]


@jax.jit
def _forward(rgb, one_hot_idx, task_obs, emb_w, emb_b,
             w_t1m, b_b1cat, w_t2m, b_b2r, w_m1, b_bm1, w_m2, b_bm2,
             w_wp1, b_bp1, w_wp2, b_bp2, w_wp3, b_bp3,
             w_wh1, b_bh1, w_wh2, b_bh2, w_wh3, b_bh3, w_whf, b_bhf):
    B = rgb.shape[0]
    Bp = _round_up(max(B, 1), TM)

    xall = jnp.concatenate(
        [rgb.reshape(B, RGB_D), task_obs.reshape(B, TASK_D)], axis=1)
    idx = one_hot_idx.astype(jnp.int32)
    if Bp != B:
        xall = jnp.pad(xall, ((0, Bp - B), (0, 0)))
        idx = jnp.pad(idx, (0, Bp - B))

    w1r = w_t1m.reshape(RF, CH, LANE).transpose(1, 0, 2).reshape(CH * RF, LANE)

    weights = (w1r, b_b1cat, w_t2m, b_b2r, w_m1, b_bm1, w_m2, b_bm2,
               w_wp1, b_bp1, w_wp2, b_bp2, w_wp3, b_bp3,
               w_wh1, b_bh1, w_wh2, b_bh2, w_wh3, b_bh3, w_whf, b_bhf)

    in_specs = [
        pl.BlockSpec((TM, RGB_D + TASK_D), lambda i: (i, 0)),
        pl.BlockSpec((TM,), lambda i: (i,)),
        pl.BlockSpec(emb_w.shape, lambda i: (0, 0)),
        pl.BlockSpec(emb_b.shape, lambda i: (0,)),
    ] + [pl.BlockSpec(w.shape, lambda i: (0, 0)) for w in weights]

    logits, values = pl.pallas_call(
        _fused_body,
        grid=(Bp // TM,),
        in_specs=in_specs,
        out_specs=[pl.BlockSpec((TM, NOUT), lambda i: (i, 0)),
                   pl.BlockSpec((TM,), lambda i: (i,))],
        out_shape=[jax.ShapeDtypeStruct((Bp, NOUT), jnp.float32),
                   jax.ShapeDtypeStruct((Bp,), jnp.float32)],
        compiler_params=pltpu.CompilerParams(
            dimension_semantics=("arbitrary",)),
    )(xall, idx, emb_w, emb_b, *weights)

    return logits[:B], values[:B]


def kernel(rgb, one_hot_idx, task_obs, emb_w, emb_b,
           w_t1m, b_b1cat, w_t2m, b_b2r, w_m1, b_bm1, w_m2, b_bm2,
           w_wp1, b_bp1, w_wp2, b_bp2, w_wp3, b_bp3,
           w_wh1, b_bh1, w_wh2, b_bh2, w_wh3, b_bh3, w_whf, b_bhf):
    return _forward(rgb, one_hot_idx, task_obs, emb_w, emb_b,
                    w_t1m, b_b1cat, w_t2m, b_b2r, w_m1, b_bm1, w_m2, b_bm2,
                    w_wp1, b_bp1, w_wp2, b_bp2, w_wp3, b_bp3,
                    w_wh1, b_bh1, w_wh2, b_bh2, w_wh3, b_bh3, w_whf, b_bhf)


# speedup vs baseline: 3.5355x; 3.5355x over previous
import jax
import jax.numpy as jnp
from jax import lax
from jax.experimental import pallas as pl
from jax.experimental.pallas import tpu as pltpu

LANE = 128
OH1 = 15
CH = 4
HW_LANES = 1024
ROW_STRIDE = 64
RF = 128
RGB_D = CH * HW_LANES
TASK_D = 80
NOUT = 64
TM = 512


def _round_up(x, m):
    return ((x + m - 1) // m) * m


def _contract0(w, x, f32):
    return lax.dot_general(w, x, (((0,), (0,)), ((), ())),
                           preferred_element_type=f32)


def _fused_body(x_ref, idx_ref, task_ref, embwT_ref, embb_ref,
                w1_ref, b1_ref, t2_ref, b2_ref,
                m1_ref, bm1_ref, m2_ref, bm2_ref,
                wp1_ref, bp1_ref, wp2_ref, bp2_ref, wp3_ref, bp3_ref,
                wh1_ref, bh1_ref, wh2_ref, bh2_ref, wh3_ref, bh3_ref,
                whf_ref, bhf_ref, logit_ref, value_ref):
    bf16 = jnp.bfloat16
    f32 = jnp.float32

    def dense(x, w_ref, b_ref, relu=True, out_dtype=bf16):
        y = _contract0(w_ref[...], x, f32) + b_ref[...].T
        if relu:
            y = jnp.maximum(y, 0.0)
        return y.astype(out_dtype)

    xb = x_ref[...].astype(bf16)
    parts = []
    for oh in range(OH1):
        off = oh * ROW_STRIDE
        lhs = jnp.concatenate(
            [xb[c * HW_LANES + off: c * HW_LANES + off + RF, :]
             for c in range(CH)],
            axis=0)
        p = _contract0(w1_ref[...], lhs, f32)
        p = p + b1_ref[:, oh * LANE:(oh + 1) * LANE].T
        parts.append(jnp.maximum(p, 0.0).astype(bf16))
    h1 = jnp.concatenate(parts, axis=0)
    cnn = dense(h1, t2_ref, b2_ref)

    onehot = (idx_ref[...][None, :] ==
              lax.broadcasted_iota(jnp.int32, (64, TM), 0)).astype(f32)
    emb = jnp.maximum(
        jnp.dot(embwT_ref[...], onehot, preferred_element_type=f32)
        + embb_ref[...][:, None], 0.0)
    a1 = (_contract0(m1_ref[0:32, :], emb.astype(bf16), f32)
          + _contract0(m1_ref[32:112, :], task_ref[...].astype(bf16), f32)
          + bm1_ref[...].T)
    a1 = jnp.maximum(a1, 0.0).astype(bf16)
    a2 = dense(a1, m2_ref, bm2_ref)

    cat = cnn + a2
    x = dense(cat, wp1_ref, bp1_ref)
    x = dense(x, wp2_ref, bp2_ref)
    x = dense(x, wp3_ref, bp3_ref)
    hh = dense(x, wh1_ref, bh1_ref)
    hh = dense(hh, wh2_ref, bh2_ref)
    hh = dense(hh, wh3_ref, bh3_ref)
    y = _contract0(whf_ref[...], hh, f32) + bhf_ref[...].T
    logit_ref[...] = y[:NOUT, :]
    value_ref[...] = y[NOUT, :]


@jax.jit
def _forward(rgb, one_hot_idx, task_obs, emb_w, emb_b,
             w_t1m, b_b1cat, w_t2m, b_b2r, w_m1, b_bm1, w_m2, b_bm2,
             w_wp1, b_bp1, w_wp2, b_bp2, w_wp3, b_bp3,
             w_wh1, b_bh1, w_wh2, b_bh2, w_wh3, b_bh3, w_whf, b_bhf):
    B = rgb.shape[0]
    Bp = _round_up(max(B, 1), TM)

    xt = rgb.reshape(B, RGB_D).T
    taskT = task_obs.reshape(B, TASK_D).T
    embwT = emb_w.T
    idx = one_hot_idx.astype(jnp.int32)
    if Bp != B:
        xt = jnp.pad(xt, ((0, 0), (0, Bp - B)))
        taskT = jnp.pad(taskT, ((0, 0), (0, Bp - B)))
        idx = jnp.pad(idx, (0, Bp - B))

    w1r = w_t1m.reshape(RF, CH, LANE).transpose(1, 0, 2).reshape(CH * RF, LANE)

    weights = (w1r, b_b1cat, w_t2m, b_b2r, w_m1, b_bm1, w_m2, b_bm2,
               w_wp1, b_bp1, w_wp2, b_bp2, w_wp3, b_bp3,
               w_wh1, b_bh1, w_wh2, b_bh2, w_wh3, b_bh3, w_whf, b_bhf)

    in_specs = [
        pl.BlockSpec((RGB_D, TM), lambda i: (0, i)),
        pl.BlockSpec((TM,), lambda i: (i,)),
        pl.BlockSpec((TASK_D, TM), lambda i: (0, i)),
        pl.BlockSpec(embwT.shape, lambda i: (0, 0)),
        pl.BlockSpec(emb_b.shape, lambda i: (0,)),
    ] + [pl.BlockSpec(w.shape, lambda i: (0, 0)) for w in weights]

    logitsT, values = pl.pallas_call(
        _fused_body,
        grid=(Bp // TM,),
        in_specs=in_specs,
        out_specs=[pl.BlockSpec((NOUT, TM), lambda i: (0, i)),
                   pl.BlockSpec((TM,), lambda i: (i,))],
        out_shape=[jax.ShapeDtypeStruct((NOUT, Bp), jnp.float32),
                   jax.ShapeDtypeStruct((Bp,), jnp.float32)],
        compiler_params=pltpu.CompilerParams(
            dimension_semantics=("arbitrary",)),
    )(xt, idx, taskT, embwT, emb_b, *weights)

    return logitsT.T[:B], values[:B]


def kernel(rgb, one_hot_idx, task_obs, emb_w, emb_b,
           w_t1m, b_b1cat, w_t2m, b_b2r, w_m1, b_bm1, w_m2, b_bm2,
           w_wp1, b_bp1, w_wp2, b_bp2, w_wp3, b_bp3,
           w_wh1, b_bh1, w_wh2, b_bh2, w_wh3, b_bh3, w_whf, b_bhf):
    return _forward(rgb, one_hot_idx, task_obs, emb_w, emb_b,
                    w_t1m, b_b1cat, w_t2m, b_b2r, w_m1, b_bm1, w_m2, b_bm2,
                    w_wp1, b_bp1, w_wp2, b_bp2, w_wp3, b_bp3,
                    w_wh1, b_bh1, w_wh2, b_bh2, w_wh3, b_bh3, w_whf, b_bhf)


# speedup vs baseline: 3.8478x; 1.0883x over previous
import jax
import jax.numpy as jnp
from jax import lax
from jax.experimental import pallas as pl
from jax.experimental.pallas import tpu as pltpu

LANE = 128
OH1 = 15
CH = 4
HW_LANES = 1024
ROW_STRIDE = 64
RF = 128
RGB_D = CH * HW_LANES
TASK_D = 80
NOUT = 64
TM = 1024


def _round_up(x, m):
    return ((x + m - 1) // m) * m


def _contract0(w, x, f32):
    return lax.dot_general(w, x, (((0,), (0,)), ((), ())),
                           preferred_element_type=f32)


def _fused_body(x_ref, idx_ref, task_ref, embwT_ref, embb_ref,
                w1_ref, b1_ref, t2_ref, b2_ref,
                m1_ref, bm1_ref, m2_ref, bm2_ref,
                wp1_ref, bp1_ref, wp2_ref, bp2_ref, wp3_ref, bp3_ref,
                wh1_ref, bh1_ref, wh2_ref, bh2_ref, wh3_ref, bh3_ref,
                whf_ref, bhf_ref, logit_ref, value_ref):
    bf16 = jnp.bfloat16
    f32 = jnp.float32

    def dense(x, w_ref, b_ref, relu=True, out_dtype=bf16):
        y = _contract0(w_ref[...], x, f32) + b_ref[...].T
        if relu:
            y = jnp.maximum(y, 0.0)
        return y.astype(out_dtype)

    xb = x_ref[...].astype(bf16)
    parts = []
    for oh in range(OH1):
        off = oh * ROW_STRIDE
        lhs = jnp.concatenate(
            [xb[c * HW_LANES + off: c * HW_LANES + off + RF, :]
             for c in range(CH)],
            axis=0)
        p = _contract0(w1_ref[...], lhs, f32)
        p = p + b1_ref[:, oh * LANE:(oh + 1) * LANE].T
        parts.append(jnp.maximum(p, 0.0).astype(bf16))
    h1 = jnp.concatenate(parts, axis=0)
    cnn = dense(h1, t2_ref, b2_ref)

    onehot = (idx_ref[...][None, :] ==
              lax.broadcasted_iota(jnp.int32, (64, TM), 0)).astype(f32)
    emb = jnp.maximum(
        jnp.dot(embwT_ref[...], onehot, preferred_element_type=f32)
        + embb_ref[...][:, None], 0.0)
    a1 = (_contract0(m1_ref[0:32, :], emb.astype(bf16), f32)
          + _contract0(m1_ref[32:112, :], task_ref[...].astype(bf16), f32)
          + bm1_ref[...].T)
    a1 = jnp.maximum(a1, 0.0).astype(bf16)
    a2 = dense(a1, m2_ref, bm2_ref)

    cat = cnn + a2
    x = dense(cat, wp1_ref, bp1_ref)
    x = dense(x, wp2_ref, bp2_ref)
    x = dense(x, wp3_ref, bp3_ref)
    hh = dense(x, wh1_ref, bh1_ref)
    hh = dense(hh, wh2_ref, bh2_ref)
    hh = dense(hh, wh3_ref, bh3_ref)
    y = _contract0(whf_ref[...], hh, f32) + bhf_ref[...].T
    logit_ref[...] = y[:NOUT, :]
    value_ref[...] = y[NOUT, :]


@jax.jit
def _forward(rgb, one_hot_idx, task_obs, emb_w, emb_b,
             w_t1m, b_b1cat, w_t2m, b_b2r, w_m1, b_bm1, w_m2, b_bm2,
             w_wp1, b_bp1, w_wp2, b_bp2, w_wp3, b_bp3,
             w_wh1, b_bh1, w_wh2, b_bh2, w_wh3, b_bh3, w_whf, b_bhf):
    B = rgb.shape[0]
    Bp = _round_up(max(B, 1), TM)

    xt = rgb.reshape(B, RGB_D).T
    taskT = task_obs.reshape(B, TASK_D).T
    embwT = emb_w.T
    idx = one_hot_idx.astype(jnp.int32)
    if Bp != B:
        xt = jnp.pad(xt, ((0, 0), (0, Bp - B)))
        taskT = jnp.pad(taskT, ((0, 0), (0, Bp - B)))
        idx = jnp.pad(idx, (0, Bp - B))

    w1r = w_t1m.reshape(RF, CH, LANE).transpose(1, 0, 2).reshape(CH * RF, LANE)

    weights = (w1r, b_b1cat, w_t2m, b_b2r, w_m1, b_bm1, w_m2, b_bm2,
               w_wp1, b_bp1, w_wp2, b_bp2, w_wp3, b_bp3,
               w_wh1, b_bh1, w_wh2, b_bh2, w_wh3, b_bh3, w_whf, b_bhf)

    in_specs = [
        pl.BlockSpec((RGB_D, TM), lambda i: (0, i)),
        pl.BlockSpec((TM,), lambda i: (i,)),
        pl.BlockSpec((TASK_D, TM), lambda i: (0, i)),
        pl.BlockSpec(embwT.shape, lambda i: (0, 0)),
        pl.BlockSpec(emb_b.shape, lambda i: (0,)),
    ] + [pl.BlockSpec(w.shape, lambda i: (0, 0)) for w in weights]

    logitsT, values = pl.pallas_call(
        _fused_body,
        grid=(Bp // TM,),
        in_specs=in_specs,
        out_specs=[pl.BlockSpec((NOUT, TM), lambda i: (0, i)),
                   pl.BlockSpec((TM,), lambda i: (i,))],
        out_shape=[jax.ShapeDtypeStruct((NOUT, Bp), jnp.float32),
                   jax.ShapeDtypeStruct((Bp,), jnp.float32)],
        compiler_params=pltpu.CompilerParams(
            dimension_semantics=("arbitrary",)),
    )(xt, idx, taskT, embwT, emb_b, *weights)

    return logitsT.T[:B], values[:B]


def kernel(rgb, one_hot_idx, task_obs, emb_w, emb_b,
           w_t1m, b_b1cat, w_t2m, b_b2r, w_m1, b_bm1, w_m2, b_bm2,
           w_wp1, b_bp1, w_wp2, b_bp2, w_wp3, b_bp3,
           w_wh1, b_bh1, w_wh2, b_bh2, w_wh3, b_bh3, w_whf, b_bhf):
    return _forward(rgb, one_hot_idx, task_obs, emb_w, emb_b,
                    w_t1m, b_b1cat, w_t2m, b_b2r, w_m1, b_bm1, w_m2, b_bm2,
                    w_wp1, b_bp1, w_wp2, b_bp2, w_wp3, b_bp3,
                    w_wh1, b_bh1, w_wh2, b_bh2, w_wh3, b_bh3, w_whf, b_bhf)
